# baseline (speedup 1.0000x reference)
import functools

import jax
import jax.numpy as jnp
from jax import lax
from jax.experimental import pallas as pl
from jax.experimental.pallas import tpu as pltpu
from jax.experimental.pallas import tpu_sc as plsc

B, S, DM = 2, 4096, 768
H, HD = 12, 64
NB, MAXB = 64, 256
BS = B * S
R = H * NB * MAXB
AB = 8
RP = R + AB * MAXB
SCALE = 1.0 / (HD ** 0.5)
CH = 1024
NCH = BS // CH
CHR = 512
NCHR = BS // CHR
PK = 2 * HD


def _bf16_hi_bits(x):
    rt = x.astype(jnp.bfloat16).astype(jnp.float32)
    return lax.bitcast_convert_type(rt, jnp.int32)


def _proj_route_body(x_ref, wq_ref, bq_ref, wk_ref, bk_ref, wv_ref, bv_ref,
                     hp_ref, qkv_ref, offs_ref, counts_ref, carry_ref):
    c = pl.program_id(0)
    dn2 = (((1,), (0,)), ((), ()))
    xb = x_ref[...]
    dn = (((1,), (1,)), ((), ()))
    yq = lax.dot_general(xb, wq_ref[...], dn) + bq_ref[...]
    xb16 = xb.astype(jnp.bfloat16)
    yk = lax.dot_general(xb16, wk_ref[...], dn,
                         preferred_element_type=jnp.float32) + bk_ref[...]
    yv = lax.dot_general(xb16, wv_ref[...], dn,
                         preferred_element_type=jnp.float32) + bv_ref[...]

    col = lax.broadcasted_iota(jnp.int32, (CHR, NB), 1).astype(jnp.float32)
    parts = []
    for h in range(H):
        hsb = jnp.dot(yq[:, h * HD:(h + 1) * HD], hp_ref[h])
        rowmax = jnp.max(hsb, axis=1, keepdims=True)
        am = jnp.min(jnp.where(hsb == rowmax, col, float(NB)),
                     axis=1, keepdims=True)
        parts.append((col == am).astype(jnp.float32))
    oneh = jnp.concatenate(parts, axis=1)

    ri = lax.broadcasted_iota(jnp.int32, (CHR, CHR), 0)
    ci = lax.broadcasted_iota(jnp.int32, (CHR, CHR), 1)
    ltri = (ri >= ci).astype(jnp.bfloat16)
    cum = lax.dot_general(ltri, oneh.astype(jnp.bfloat16), dn2,
                          preferred_element_type=jnp.float32)

    for h in range(H):
        oh = oneh[:, h * HD:(h + 1) * HD]
        cm = cum[:, h * HD:(h + 1) * HD]
        carry = jnp.where(c == 0, 0.0, carry_ref[h:h + 1, :])
        pos = jnp.sum((cm - 1.0 + carry) * oh, axis=1)
        b = jnp.sum(col * oh, axis=1)
        glob = (b + h * NB) * MAXB + pos
        offs_ref[0, h, :] = jnp.where(pos < MAXB, glob,
                                      float(R)).astype(jnp.int32)
        new_carry = carry + cm[CHR - 1:CHR, :]
        carry_ref[h:h + 1, :] = new_carry
        counts_ref[h:h + 1, :, :] = new_carry.reshape(1, 1, NB)

    for h in range(H):
        sl = slice(h * HD, (h + 1) * HD)
        hi = _bf16_hi_bits(jnp.concatenate([yq[:, sl], yv[:, sl]], axis=1))
        lo = _bf16_hi_bits(jnp.concatenate([yk[:, sl], yk[:, sl]], axis=1))
        qkv_ref[h] = hi | lax.shift_right_logical(lo, 16)


def _proj_route(xf, Wq, bq, Wk, bk, Wv, bv, hash_proj):
    wspec = pl.BlockSpec((DM, DM), lambda c: (0, 0))
    wspec16 = pl.BlockSpec((DM, DM), lambda c: (0, 0))
    bspec = pl.BlockSpec((1, DM), lambda c: (0, 0))
    rspec = pl.BlockSpec((CHR, DM), lambda c: (c, 0))
    return pl.pallas_call(
        _proj_route_body,
        grid=(NCHR,),
        in_specs=[rspec, wspec, bspec, wspec16, bspec, wspec16, bspec,
                  pl.BlockSpec((H, HD, NB), lambda c: (0, 0, 0))],
        out_specs=[
            pl.BlockSpec((H, CHR, PK), lambda c: (0, c, 0)),
            pl.BlockSpec((1, H, CHR), lambda c: (c, 0, 0)),
            pl.BlockSpec((H, 1, NB), lambda c: (0, 0, 0)),
        ],
        out_shape=[
            jax.ShapeDtypeStruct((H, BS, PK), jnp.int32),
            jax.ShapeDtypeStruct((NCHR, H, CHR), jnp.int32),
            jax.ShapeDtypeStruct((H, 1, NB), jnp.float32),
        ],
        scratch_shapes=[pltpu.VMEM((H, NB), jnp.float32)],
    )(xf, Wq, bq.reshape(1, DM), Wk.astype(jnp.bfloat16), bk.reshape(1, DM),
      Wv.astype(jnp.bfloat16), bv.reshape(1, DM), hash_proj)


def _sc_scatter(qkvh, offs_flat):
    mesh = plsc.VectorSubcoreMesh(core_axis_name="c", subcore_axis_name="s")

    @functools.partial(
        pl.kernel,
        out_type=jax.ShapeDtypeStruct((RP, PK), jnp.int32),
        mesh=mesh,
        scratch_types=[
            pltpu.VMEM((2, MAXB, PK), jnp.int32),
            pltpu.VMEM((2 * H, 128), jnp.int32),
            pltpu.SemaphoreType.DMA,
            pltpu.SemaphoreType.DMA,
            pltpu.SemaphoreType.DMA,
            pltpu.SemaphoreType.DMA,
            pltpu.SemaphoreType.DMA,
        ],
    )
    def run(qkv_hbm, offs_hbm, blk_hbm, rows_v, idx_v, sem_i,
            sem_s0, sem_s1, sem_c0, sem_c1):
        wid = lax.axis_index("s") * 2 + lax.axis_index("c")
        t0 = wid * MAXB
        base0 = (t0 // CHR) * (H * CHR) + (t0 % CHR)
        sem_s = (sem_s0, sem_s1)
        sem_c = (sem_c0, sem_c1)

        iw = []
        for h in range(H):
            for j in range(2):
                iw.append(pltpu.async_copy(
                    offs_hbm.at[pl.ds(base0 + h * CHR + j * 128, 128)],
                    idx_v.at[2 * h + j], sem_i))
        for w in iw:
            w.wait()

        def stage(h, b):
            return pltpu.async_copy(qkv_hbm.at[h, pl.ds(t0, MAXB)],
                                    rows_v.at[b], sem_c[b])

        cw = stage(0, 0)
        sw = []
        for h in range(H):
            b = h % 2
            cw.wait()
            if h >= 1:
                for w in sw:
                    w.wait()
            if h < H - 1:
                cw = stage(h + 1, 1 - b)
            sw = [pltpu.async_copy(rows_v.at[b, pl.ds(j * 128, 128)],
                                   blk_hbm.at[idx_v.at[2 * h + j]],
                                   sem_s[b])
                  for j in range(2)]
        for w in sw:
            w.wait()

    return run(qkvh, offs_flat)


def _attn_body(cnt_ref, blk_ref, out_ref):
    g = pl.program_id(0)

    @pl.when(g == (H * NB) // AB)
    def _zero():
        out_ref[...] = jnp.zeros_like(out_ref)

    @pl.when(g < (H * NB) // AB)
    def _attend():
        rowi = lax.broadcasted_iota(jnp.int32, (MAXB, HD), 0).astype(
            jnp.float32)
        coli = lax.broadcasted_iota(jnp.int32, (MAXB, MAXB), 1).astype(
            jnp.float32)
        for i in range(AB):
            cnt = cnt_ref[g * AB + i]
            blk = blk_ref[i * MAXB:(i + 1) * MAXB, :]
            hi = lax.bitcast_convert_type(
                jnp.bitwise_and(blk, jnp.int32(-65536)), jnp.float32)
            lo = lax.bitcast_convert_type(
                lax.shift_left(blk, 16), jnp.float32)
            q = hi[:, 0:HD].astype(jnp.bfloat16)
            v = hi[:, HD:2 * HD]
            k = lo[:, 0:HD].astype(jnp.bfloat16)
            dots = lax.dot_general(q, k, (((1,), (1,)), ((), ())),
                                   preferred_element_type=jnp.float32)
            dots = dots * SCALE
            dots = jnp.where(coli < cnt, dots, -jnp.inf)
            m = jnp.max(dots, axis=1, keepdims=True)
            e = jnp.exp(dots - m)
            s = jnp.sum(e, axis=1, keepdims=True)
            attnw = (e / s).astype(jnp.bfloat16)
            vb = jnp.where(rowi < cnt, v, 0.0).astype(jnp.bfloat16)
            res = lax.dot_general(attnw, vb, (((1,), (0,)), ((), ())),
                                  preferred_element_type=jnp.float32)
            out_ref[i * MAXB:(i + 1) * MAXB, :] = jnp.concatenate(
                [res, jnp.zeros((MAXB, HD), jnp.float32)], axis=1)


def _attn(cnt, qkv_blk):
    return pl.pallas_call(
        _attn_body,
        grid=((H * NB) // AB + 1,),
        in_specs=[pl.BlockSpec(memory_space=pltpu.SMEM),
                  pl.BlockSpec((AB * MAXB, PK), lambda g: (g, 0))],
        out_specs=pl.BlockSpec((AB * MAXB, 2 * HD), lambda g: (g, 0)),
        out_shape=jax.ShapeDtypeStruct((RP, 2 * HD), jnp.float32),
    )(cnt, qkv_blk)


def _sc_gather(res, offs_flat):
    mesh = plsc.VectorSubcoreMesh(core_axis_name="c", subcore_axis_name="s")

    @functools.partial(
        pl.kernel,
        out_type=jax.ShapeDtypeStruct((H, BS, 2 * HD), jnp.float32),
        mesh=mesh,
        scratch_types=[
            pltpu.VMEM((2, 128, 2 * HD), jnp.float32),
            pltpu.VMEM((2 * H, 128), jnp.int32),
            pltpu.SemaphoreType.DMA,
            pltpu.SemaphoreType.DMA,
            pltpu.SemaphoreType.DMA,
            pltpu.SemaphoreType.DMA,
            pltpu.SemaphoreType.DMA,
        ],
    )
    def run(res_hbm, offs_hbm, out_hbm, gbuf, idxg, sem_i,
            sem_g0, sem_g1, sem_w0, sem_w1):
        wid = lax.axis_index("s") * 2 + lax.axis_index("c")
        sem_g = (sem_g0, sem_g1)
        sem_w = (sem_w0, sem_w1)
        units = [((wid * 2 + half) * 128, h)
                 for half in range(2) for h in range(H)]

        iw = []
        for u, (t0, h) in enumerate(units):
            base = (t0 // CHR) * (H * CHR) + (t0 % CHR) + h * CHR
            iw.append(pltpu.async_copy(offs_hbm.at[pl.ds(base, 128)],
                                       idxg.at[u], sem_i))
        for w in iw:
            w.wait()

        def fire(u, b):
            return pltpu.async_copy(res_hbm.at[idxg.at[u]], gbuf.at[b],
                                    sem_g[b])

        gw = fire(0, 0)
        ww = None
        for u, (t0, h) in enumerate(units):
            b = u % 2
            gw.wait()
            if ww is not None:
                ww.wait()
            if u < len(units) - 1:
                gw = fire(u + 1, 1 - b)
            ww = pltpu.async_copy(gbuf.at[b],
                                  out_hbm.at[h, pl.ds(t0, 128)], sem_w[b])
        ww.wait()

    return run(res, offs_flat)


def _wo_body(oa_ref, w_ref, b_ref, o_ref):
    a = jnp.concatenate([oa_ref[h][:, :HD] for h in range(H)],
                        axis=1).astype(jnp.bfloat16)
    dn = (((1,), (1,)), ((), ()))
    o_ref[...] = lax.dot_general(a, w_ref[...], dn,
                                 preferred_element_type=jnp.float32
                                 ) + b_ref[...]


def _wo(oa, Wo, bo):
    return pl.pallas_call(
        _wo_body,
        grid=(NCH,),
        in_specs=[pl.BlockSpec((H, CH, 2 * HD), lambda c: (0, c, 0)),
                  pl.BlockSpec((DM, DM), lambda c: (0, 0)),
                  pl.BlockSpec((1, DM), lambda c: (0, 0))],
        out_specs=pl.BlockSpec((CH, DM), lambda c: (c, 0)),
        out_shape=jax.ShapeDtypeStruct((BS, DM), jnp.float32),
    )(oa, Wo.astype(jnp.bfloat16), bo.reshape(1, DM))


def kernel(x, Wq, bq, Wk, bk, Wv, bv, Wo, bo, hash_proj):
    xf = x.reshape(BS, DM)
    qkvh, offs, counts = _proj_route(xf, Wq, bq, Wk, bk, Wv, bv, hash_proj)
    offs_flat = offs.reshape(NCHR * H * CHR)
    cnt = counts.reshape(H * NB)
    qkv_blk = _sc_scatter(qkvh, offs_flat)
    res = _attn(cnt, qkv_blk)
    oa = _sc_gather(res, offs_flat)
    out = _wo(oa, Wo, bo)
    return out.reshape(B, S, DM)


# speedup vs baseline: 1.0925x; 1.0925x over previous
import functools

import jax
import jax.numpy as jnp
from jax import lax
from jax.experimental import pallas as pl
from jax.experimental.pallas import tpu as pltpu
from jax.experimental.pallas import tpu_sc as plsc

B, S, DM = 2, 4096, 768
H, HD = 12, 64
NB, MAXB = 64, 256
BS = B * S
R = H * NB * MAXB
AB = 8
HG = H // 2
RG = HG * NB * MAXB
RGP = RG + AB * MAXB
SCALE = 1.0 / (HD ** 0.5)
CH = 1024
NCH = BS // CH
CHR = 512
NCHR = BS // CHR
PK = 2 * HD


def _bf16_hi_bits(x):
    rt = x.astype(jnp.bfloat16).astype(jnp.float32)
    return lax.bitcast_convert_type(rt, jnp.int32)


def _proj_route_body(x_ref, wq_ref, bq_ref, wk_ref, bk_ref, wv_ref, bv_ref,
                     hp_ref, qkv_ref, offs_ref, counts_ref, carry_ref):
    c = pl.program_id(0)
    dn2 = (((1,), (0,)), ((), ()))
    xb = x_ref[...]
    dn = (((1,), (1,)), ((), ()))
    yq = lax.dot_general(xb, wq_ref[...], dn) + bq_ref[...]
    xb16 = xb.astype(jnp.bfloat16)
    yk = lax.dot_general(xb16, wk_ref[...], dn,
                         preferred_element_type=jnp.float32) + bk_ref[...]
    yv = lax.dot_general(xb16, wv_ref[...], dn,
                         preferred_element_type=jnp.float32) + bv_ref[...]

    col = lax.broadcasted_iota(jnp.int32, (CHR, NB), 1).astype(jnp.float32)
    parts = []
    for h in range(H):
        hsb = jnp.dot(yq[:, h * HD:(h + 1) * HD], hp_ref[h])
        rowmax = jnp.max(hsb, axis=1, keepdims=True)
        am = jnp.min(jnp.where(hsb == rowmax, col, float(NB)),
                     axis=1, keepdims=True)
        parts.append((col == am).astype(jnp.float32))
    oneh = jnp.concatenate(parts, axis=1)

    ri = lax.broadcasted_iota(jnp.int32, (CHR, CHR), 0)
    ci = lax.broadcasted_iota(jnp.int32, (CHR, CHR), 1)
    ltri = (ri >= ci).astype(jnp.bfloat16)
    cum = lax.dot_general(ltri, oneh.astype(jnp.bfloat16), dn2,
                          preferred_element_type=jnp.float32)

    for h in range(H):
        oh = oneh[:, h * HD:(h + 1) * HD]
        cm = cum[:, h * HD:(h + 1) * HD]
        carry = jnp.where(c == 0, 0.0, carry_ref[h:h + 1, :])
        pos = jnp.sum((cm - 1.0 + carry) * oh, axis=1)
        b = jnp.sum(col * oh, axis=1)
        glob = (b + (h % HG) * NB) * MAXB + pos
        offs_ref[0, h, :] = jnp.where(pos < MAXB, glob,
                                      float(RG)).astype(jnp.int32)
        new_carry = carry + cm[CHR - 1:CHR, :]
        carry_ref[h:h + 1, :] = new_carry
        counts_ref[h:h + 1, :, :] = new_carry.reshape(1, 1, NB)

    for h in range(H):
        sl = slice(h * HD, (h + 1) * HD)
        hi = _bf16_hi_bits(jnp.concatenate([yq[:, sl], yv[:, sl]], axis=1))
        lo = _bf16_hi_bits(jnp.concatenate([yk[:, sl], yk[:, sl]], axis=1))
        qkv_ref[h] = hi | lax.shift_right_logical(lo, 16)


def _proj_route(xf, Wq, bq, Wk, bk, Wv, bv, hash_proj):
    wspec = pl.BlockSpec((DM, DM), lambda c: (0, 0))
    wspec16 = pl.BlockSpec((DM, DM), lambda c: (0, 0))
    bspec = pl.BlockSpec((1, DM), lambda c: (0, 0))
    rspec = pl.BlockSpec((CHR, DM), lambda c: (c, 0))
    return pl.pallas_call(
        _proj_route_body,
        grid=(NCHR,),
        in_specs=[rspec, wspec, bspec, wspec16, bspec, wspec16, bspec,
                  pl.BlockSpec((H, HD, NB), lambda c: (0, 0, 0))],
        out_specs=[
            pl.BlockSpec((H, CHR, PK), lambda c: (0, c, 0)),
            pl.BlockSpec((1, H, CHR), lambda c: (c, 0, 0)),
            pl.BlockSpec((H, 1, NB), lambda c: (0, 0, 0)),
        ],
        out_shape=[
            jax.ShapeDtypeStruct((H, BS, PK), jnp.int32),
            jax.ShapeDtypeStruct((NCHR, H, CHR), jnp.int32),
            jax.ShapeDtypeStruct((H, 1, NB), jnp.float32),
        ],
        scratch_shapes=[pltpu.VMEM((H, NB), jnp.float32)],
    )(xf, Wq, bq.reshape(1, DM), Wk.astype(jnp.bfloat16), bk.reshape(1, DM),
      Wv.astype(jnp.bfloat16), bv.reshape(1, DM), hash_proj)


def _sc_scatter(qkvh, offs_flat, h0):
    mesh = plsc.VectorSubcoreMesh(core_axis_name="c", subcore_axis_name="s")

    @functools.partial(
        pl.kernel,
        out_type=jax.ShapeDtypeStruct((RGP, PK), jnp.int32),
        mesh=mesh,
        scratch_types=[
            pltpu.VMEM((2, MAXB, PK), jnp.int32),
            pltpu.VMEM((2 * HG, 128), jnp.int32),
            pltpu.SemaphoreType.DMA,
            pltpu.SemaphoreType.DMA,
            pltpu.SemaphoreType.DMA,
            pltpu.SemaphoreType.DMA,
            pltpu.SemaphoreType.DMA,
        ],
    )
    def run(qkv_hbm, offs_hbm, blk_hbm, rows_v, idx_v, sem_i,
            sem_s0, sem_s1, sem_c0, sem_c1):
        wid = lax.axis_index("s") * 2 + lax.axis_index("c")
        t0 = wid * MAXB
        base0 = (t0 // CHR) * (H * CHR) + (t0 % CHR)
        sem_s = (sem_s0, sem_s1)
        sem_c = (sem_c0, sem_c1)

        iw = []
        for u, h in enumerate(range(h0, h0 + HG)):
            for j in range(2):
                iw.append(pltpu.async_copy(
                    offs_hbm.at[pl.ds(base0 + h * CHR + j * 128, 128)],
                    idx_v.at[2 * u + j], sem_i))
        for w in iw:
            w.wait()

        def stage(h, b):
            return pltpu.async_copy(qkv_hbm.at[h, pl.ds(t0, MAXB)],
                                    rows_v.at[b], sem_c[b])

        cw = stage(h0, 0)
        sw = []
        for u, h in enumerate(range(h0, h0 + HG)):
            b = u % 2
            cw.wait()
            if u >= 1:
                for w in sw:
                    w.wait()
            if u < HG - 1:
                cw = stage(h + 1, 1 - b)
            sw = [pltpu.async_copy(rows_v.at[b, pl.ds(j * 128, 128)],
                                   blk_hbm.at[idx_v.at[2 * u + j]],
                                   sem_s[b])
                  for j in range(2)]
        for w in sw:
            w.wait()

    return run(qkvh, offs_flat)


def _attn_body(cnt_ref, blk_ref, out_ref):
    g = pl.program_id(0)

    @pl.when(g == (HG * NB) // AB)
    def _zero():
        out_ref[...] = jnp.zeros_like(out_ref)

    @pl.when(g < (HG * NB) // AB)
    def _attend():
        rowi = lax.broadcasted_iota(jnp.int32, (MAXB, HD), 0).astype(
            jnp.float32)
        coli = lax.broadcasted_iota(jnp.int32, (MAXB, MAXB), 1).astype(
            jnp.float32)
        for i in range(AB):
            cnt = cnt_ref[g * AB + i]
            blk = blk_ref[i * MAXB:(i + 1) * MAXB, :]
            hi = lax.bitcast_convert_type(
                jnp.bitwise_and(blk, jnp.int32(-65536)), jnp.float32)
            lo = lax.bitcast_convert_type(
                lax.shift_left(blk, 16), jnp.float32)
            q = hi[:, 0:HD].astype(jnp.bfloat16)
            v = hi[:, HD:2 * HD]
            k = lo[:, 0:HD].astype(jnp.bfloat16)
            dots = lax.dot_general(q, k, (((1,), (1,)), ((), ())),
                                   preferred_element_type=jnp.float32)
            dots = dots * SCALE
            dots = jnp.where(coli < cnt, dots, -jnp.inf)
            m = jnp.max(dots, axis=1, keepdims=True)
            e = jnp.exp(dots - m)
            s = jnp.sum(e, axis=1, keepdims=True)
            attnw = (e / s).astype(jnp.bfloat16)
            vb = jnp.where(rowi < cnt, v, 0.0).astype(jnp.bfloat16)
            res = lax.dot_general(attnw, vb, (((1,), (0,)), ((), ())),
                                  preferred_element_type=jnp.float32)
            out_ref[i * MAXB:(i + 1) * MAXB, :] = jnp.concatenate(
                [res, jnp.zeros((MAXB, HD), jnp.float32)], axis=1)


def _attn(cnt, qkv_blk):
    return pl.pallas_call(
        _attn_body,
        grid=((HG * NB) // AB + 1,),
        in_specs=[pl.BlockSpec(memory_space=pltpu.SMEM),
                  pl.BlockSpec((AB * MAXB, PK), lambda g: (g, 0))],
        out_specs=pl.BlockSpec((AB * MAXB, 2 * HD), lambda g: (g, 0)),
        out_shape=jax.ShapeDtypeStruct((RGP, 2 * HD), jnp.float32),
    )(cnt, qkv_blk)


def _sc_gather(res, offs_flat, h0):
    mesh = plsc.VectorSubcoreMesh(core_axis_name="c", subcore_axis_name="s")

    @functools.partial(
        pl.kernel,
        out_type=jax.ShapeDtypeStruct((HG, BS, 2 * HD), jnp.float32),
        mesh=mesh,
        scratch_types=[
            pltpu.VMEM((2, 128, 2 * HD), jnp.float32),
            pltpu.VMEM((2 * HG, 128), jnp.int32),
            pltpu.SemaphoreType.DMA,
            pltpu.SemaphoreType.DMA,
            pltpu.SemaphoreType.DMA,
            pltpu.SemaphoreType.DMA,
            pltpu.SemaphoreType.DMA,
        ],
    )
    def run(res_hbm, offs_hbm, out_hbm, gbuf, idxg, sem_i,
            sem_g0, sem_g1, sem_w0, sem_w1):
        wid = lax.axis_index("s") * 2 + lax.axis_index("c")
        sem_g = (sem_g0, sem_g1)
        sem_w = (sem_w0, sem_w1)
        units = [((wid * 2 + half) * 128, h)
                 for half in range(2) for h in range(h0, h0 + HG)]

        iw = []
        for u, (t0, h) in enumerate(units):
            base = (t0 // CHR) * (H * CHR) + (t0 % CHR) + h * CHR
            iw.append(pltpu.async_copy(offs_hbm.at[pl.ds(base, 128)],
                                       idxg.at[u], sem_i))
        for w in iw:
            w.wait()

        def fire(u, b):
            return pltpu.async_copy(res_hbm.at[idxg.at[u]], gbuf.at[b],
                                    sem_g[b])

        gw = fire(0, 0)
        ww = None
        for u, (t0, h) in enumerate(units):
            b = u % 2
            gw.wait()
            if ww is not None:
                ww.wait()
            if u < len(units) - 1:
                gw = fire(u + 1, 1 - b)
            ww = pltpu.async_copy(gbuf.at[b],
                                  out_hbm.at[h - h0, pl.ds(t0, 128)],
                                  sem_w[b])
        ww.wait()

    return run(res, offs_flat)


def _wo_body(oa_ref, ob_ref, w_ref, b_ref, o_ref):
    a = jnp.concatenate([oa_ref[h][:, :HD] for h in range(HG)]
                        + [ob_ref[h][:, :HD] for h in range(HG)],
                        axis=1).astype(jnp.bfloat16)
    dn = (((1,), (1,)), ((), ()))
    o_ref[...] = lax.dot_general(a, w_ref[...], dn,
                                 preferred_element_type=jnp.float32
                                 ) + b_ref[...]


def _wo(oa, ob, Wo, bo):
    gspec = pl.BlockSpec((HG, CH, 2 * HD), lambda c: (0, c, 0))
    return pl.pallas_call(
        _wo_body,
        grid=(NCH,),
        in_specs=[gspec, gspec,
                  pl.BlockSpec((DM, DM), lambda c: (0, 0)),
                  pl.BlockSpec((1, DM), lambda c: (0, 0))],
        out_specs=pl.BlockSpec((CH, DM), lambda c: (c, 0)),
        out_shape=jax.ShapeDtypeStruct((BS, DM), jnp.float32),
    )(oa, ob, Wo.astype(jnp.bfloat16), bo.reshape(1, DM))


def kernel(x, Wq, bq, Wk, bk, Wv, bv, Wo, bo, hash_proj):
    xf = x.reshape(BS, DM)
    qkvh, offs, counts = _proj_route(xf, Wq, bq, Wk, bk, Wv, bv, hash_proj)
    offs_flat = offs.reshape(NCHR * H * CHR)
    cnt = counts.reshape(H * NB)
    blk_a = _sc_scatter(qkvh, offs_flat, 0)
    blk_b = _sc_scatter(qkvh, offs_flat, HG)
    res_a = _attn(cnt[:HG * NB], blk_a)
    res_b = _attn(cnt[HG * NB:], blk_b)
    oa = _sc_gather(res_a, offs_flat, 0)
    ob = _sc_gather(res_b, offs_flat, HG)
    out = _wo(oa, ob, Wo, bo)
    return out.reshape(B, S, DM)


# speedup vs baseline: 1.1613x; 1.0629x over previous
import functools

import jax
import jax.numpy as jnp
from jax import lax
from jax.experimental import pallas as pl
from jax.experimental.pallas import tpu as pltpu
from jax.experimental.pallas import tpu_sc as plsc

B, S, DM = 2, 4096, 768
H, HD = 12, 64
NB, MAXB = 64, 256
BS = B * S
R = H * NB * MAXB
AB = 8
HG = H // 2
RG = HG * NB * MAXB
RGP = RG + AB * MAXB
SCALE = 1.0 / (HD ** 0.5)
CH = 1024
NCH = BS // CH
CHR = 512
NCHR = BS // CHR
PK = 2 * HD


def _bf16_hi_bits(x):
    rt = x.astype(jnp.bfloat16).astype(jnp.float32)
    return lax.bitcast_convert_type(rt, jnp.int32)


def _proj_route_body(x_ref, wq_ref, bq_ref, wk_ref, bk_ref, wv_ref, bv_ref,
                     hp_ref, qkv_ref, offs_ref, counts_ref, carry_ref):
    c = pl.program_id(0)
    dn2 = (((1,), (0,)), ((), ()))
    xb = x_ref[...]
    dn = (((1,), (1,)), ((), ()))
    yq = lax.dot_general(xb, wq_ref[...], dn) + bq_ref[...]
    xb16 = xb.astype(jnp.bfloat16)
    yk = lax.dot_general(xb16, wk_ref[...], dn,
                         preferred_element_type=jnp.float32) + bk_ref[...]
    yv = lax.dot_general(xb16, wv_ref[...], dn,
                         preferred_element_type=jnp.float32) + bv_ref[...]

    col = lax.broadcasted_iota(jnp.int32, (CHR, NB), 1).astype(jnp.float32)
    parts = []
    for h in range(H):
        hsb = jnp.dot(yq[:, h * HD:(h + 1) * HD], hp_ref[h])
        rowmax = jnp.max(hsb, axis=1, keepdims=True)
        am = jnp.min(jnp.where(hsb == rowmax, col, float(NB)),
                     axis=1, keepdims=True)
        parts.append((col == am).astype(jnp.float32))
    oneh = jnp.concatenate(parts, axis=1)

    ri = lax.broadcasted_iota(jnp.int32, (CHR, CHR), 0)
    ci = lax.broadcasted_iota(jnp.int32, (CHR, CHR), 1)
    ltri = (ri >= ci).astype(jnp.bfloat16)
    cum = lax.dot_general(ltri, oneh.astype(jnp.bfloat16), dn2,
                          preferred_element_type=jnp.float32)

    for h in range(H):
        oh = oneh[:, h * HD:(h + 1) * HD]
        cm = cum[:, h * HD:(h + 1) * HD]
        carry = jnp.where(c == 0, 0.0, carry_ref[h:h + 1, :])
        pos = jnp.sum((cm - 1.0 + carry) * oh, axis=1)
        b = jnp.sum(col * oh, axis=1)
        glob = (b + (h % HG) * NB) * MAXB + pos
        offs_ref[0, h, :] = jnp.where(pos < MAXB, glob,
                                      float(RG)).astype(jnp.int32)
        new_carry = carry + cm[CHR - 1:CHR, :]
        carry_ref[h:h + 1, :] = new_carry
        counts_ref[h:h + 1, :, :] = new_carry.reshape(1, 1, NB)

    for h in range(H):
        sl = slice(h * HD, (h + 1) * HD)
        hi = _bf16_hi_bits(jnp.concatenate([yq[:, sl], yv[:, sl]], axis=1))
        lo = _bf16_hi_bits(jnp.concatenate([yk[:, sl], yk[:, sl]], axis=1))
        qkv_ref[h] = hi | lax.shift_right_logical(lo, 16)


def _proj_route(xf, Wq, bq, Wk, bk, Wv, bv, hash_proj):
    wspec = pl.BlockSpec((DM, DM), lambda c: (0, 0))
    wspec16 = pl.BlockSpec((DM, DM), lambda c: (0, 0))
    bspec = pl.BlockSpec((1, DM), lambda c: (0, 0))
    rspec = pl.BlockSpec((CHR, DM), lambda c: (c, 0))
    return pl.pallas_call(
        _proj_route_body,
        grid=(NCHR,),
        in_specs=[rspec, wspec, bspec, wspec16, bspec, wspec16, bspec,
                  pl.BlockSpec((H, HD, NB), lambda c: (0, 0, 0))],
        out_specs=[
            pl.BlockSpec((H, CHR, PK), lambda c: (0, c, 0)),
            pl.BlockSpec((1, H, CHR), lambda c: (c, 0, 0)),
            pl.BlockSpec((H, 1, NB), lambda c: (0, 0, 0)),
        ],
        out_shape=[
            jax.ShapeDtypeStruct((H, BS, PK), jnp.int32),
            jax.ShapeDtypeStruct((NCHR, H, CHR), jnp.int32),
            jax.ShapeDtypeStruct((H, 1, NB), jnp.float32),
        ],
        scratch_shapes=[pltpu.VMEM((H, NB), jnp.float32)],
    )(xf, Wq, bq.reshape(1, DM), Wk.astype(jnp.bfloat16), bk.reshape(1, DM),
      Wv.astype(jnp.bfloat16), bv.reshape(1, DM), hash_proj)


def _sc_scatter(qkvh, offs_flat, h0):
    mesh = plsc.VectorSubcoreMesh(core_axis_name="c", subcore_axis_name="s")

    @functools.partial(
        pl.kernel,
        out_type=jax.ShapeDtypeStruct((RGP, PK), jnp.int32),
        mesh=mesh,
        scratch_types=[
            pltpu.VMEM((2, MAXB, PK), jnp.int32),
            pltpu.VMEM((2 * HG, 128), jnp.int32),
            pltpu.SemaphoreType.DMA,
            pltpu.SemaphoreType.DMA,
            pltpu.SemaphoreType.DMA,
            pltpu.SemaphoreType.DMA,
            pltpu.SemaphoreType.DMA,
        ],
    )
    def run(qkv_hbm, offs_hbm, blk_hbm, rows_v, idx_v, sem_i,
            sem_s0, sem_s1, sem_c0, sem_c1):
        wid = lax.axis_index("s") * 2 + lax.axis_index("c")
        t0 = wid * MAXB
        base0 = (t0 // CHR) * (H * CHR) + (t0 % CHR)
        sem_s = (sem_s0, sem_s1)
        sem_c = (sem_c0, sem_c1)

        iw = []
        for u, h in enumerate(range(h0, h0 + HG)):
            for j in range(2):
                iw.append(pltpu.async_copy(
                    offs_hbm.at[pl.ds(base0 + h * CHR + j * 128, 128)],
                    idx_v.at[2 * u + j], sem_i))
        for w in iw:
            w.wait()

        def stage(h, b):
            return pltpu.async_copy(qkv_hbm.at[h, pl.ds(t0, MAXB)],
                                    rows_v.at[b], sem_c[b])

        cw = stage(h0, 0)
        sw = []
        for u, h in enumerate(range(h0, h0 + HG)):
            b = u % 2
            cw.wait()
            if u >= 1:
                for w in sw:
                    w.wait()
            if u < HG - 1:
                cw = stage(h + 1, 1 - b)
            sw = [pltpu.async_copy(rows_v.at[b, pl.ds(j * 128, 128)],
                                   blk_hbm.at[idx_v.at[2 * u + j]],
                                   sem_s[b])
                  for j in range(2)]
        for w in sw:
            w.wait()

    return run(qkvh, offs_flat)


def _attn_body(cnt_ref, blk_ref, out_ref):
    g = pl.program_id(0)

    @pl.when(g == (HG * NB) // AB)
    def _zero():
        out_ref[...] = jnp.zeros_like(out_ref)

    @pl.when(g < (HG * NB) // AB)
    def _attend():
        rowi = lax.broadcasted_iota(jnp.int32, (MAXB, HD), 0).astype(
            jnp.float32)
        coli = lax.broadcasted_iota(jnp.int32, (MAXB, MAXB), 1).astype(
            jnp.float32)
        for i in range(AB):
            cnt = cnt_ref[g * AB + i]
            blk = blk_ref[i * MAXB:(i + 1) * MAXB, :]
            hi = lax.bitcast_convert_type(
                jnp.bitwise_and(blk, jnp.int32(-65536)), jnp.float32)
            lo = lax.bitcast_convert_type(
                lax.shift_left(blk, 16), jnp.float32)
            q = hi[:, 0:HD].astype(jnp.bfloat16)
            v = hi[:, HD:2 * HD]
            k = lo[:, 0:HD].astype(jnp.bfloat16)
            dots = lax.dot_general(q, k, (((1,), (1,)), ((), ())),
                                   preferred_element_type=jnp.float32)
            dots = dots * SCALE
            dots = jnp.where(coli < cnt, dots, -jnp.inf)
            e = jnp.exp(dots)
            s = jnp.sum(e, axis=1, keepdims=True)
            attnw = (e / s).astype(jnp.bfloat16)
            vb = jnp.where(rowi < cnt, v, 0.0).astype(jnp.bfloat16)
            res = lax.dot_general(attnw, vb, (((1,), (0,)), ((), ())),
                                  preferred_element_type=jnp.float32)
            out_ref[i * MAXB:(i + 1) * MAXB, :] = jnp.concatenate(
                [res, jnp.zeros((MAXB, HD), jnp.float32)], axis=1)


def _attn(cnt, qkv_blk):
    return pl.pallas_call(
        _attn_body,
        grid=((HG * NB) // AB + 1,),
        in_specs=[pl.BlockSpec(memory_space=pltpu.SMEM),
                  pl.BlockSpec((AB * MAXB, PK), lambda g: (g, 0))],
        out_specs=pl.BlockSpec((AB * MAXB, 2 * HD), lambda g: (g, 0)),
        out_shape=jax.ShapeDtypeStruct((RGP, 2 * HD), jnp.float32),
    )(cnt, qkv_blk)


def _sc_gather(res, offs_flat, h0):
    mesh = plsc.VectorSubcoreMesh(core_axis_name="c", subcore_axis_name="s")

    @functools.partial(
        pl.kernel,
        out_type=jax.ShapeDtypeStruct((HG, BS, 2 * HD), jnp.float32),
        mesh=mesh,
        scratch_types=[
            pltpu.VMEM((2, 128, 2 * HD), jnp.float32),
            pltpu.VMEM((2 * HG, 128), jnp.int32),
            pltpu.SemaphoreType.DMA,
            pltpu.SemaphoreType.DMA,
            pltpu.SemaphoreType.DMA,
            pltpu.SemaphoreType.DMA,
            pltpu.SemaphoreType.DMA,
        ],
    )
    def run(res_hbm, offs_hbm, out_hbm, gbuf, idxg, sem_i,
            sem_g0, sem_g1, sem_w0, sem_w1):
        wid = lax.axis_index("s") * 2 + lax.axis_index("c")
        sem_g = (sem_g0, sem_g1)
        sem_w = (sem_w0, sem_w1)
        units = [((wid * 2 + half) * 128, h)
                 for half in range(2) for h in range(h0, h0 + HG)]

        iw = []
        for u, (t0, h) in enumerate(units):
            base = (t0 // CHR) * (H * CHR) + (t0 % CHR) + h * CHR
            iw.append(pltpu.async_copy(offs_hbm.at[pl.ds(base, 128)],
                                       idxg.at[u], sem_i))
        for w in iw:
            w.wait()

        def fire(u, b):
            return pltpu.async_copy(res_hbm.at[idxg.at[u]], gbuf.at[b],
                                    sem_g[b])

        gw = fire(0, 0)
        ww = None
        for u, (t0, h) in enumerate(units):
            b = u % 2
            gw.wait()
            if ww is not None:
                ww.wait()
            if u < len(units) - 1:
                gw = fire(u + 1, 1 - b)
            ww = pltpu.async_copy(gbuf.at[b],
                                  out_hbm.at[h - h0, pl.ds(t0, 128)],
                                  sem_w[b])
        ww.wait()

    return run(res, offs_flat)


def _wo_body(oa_ref, ob_ref, w_ref, b_ref, o_ref):
    a = jnp.concatenate([oa_ref[h][:, :HD] for h in range(HG)]
                        + [ob_ref[h][:, :HD] for h in range(HG)],
                        axis=1).astype(jnp.bfloat16)
    dn = (((1,), (1,)), ((), ()))
    o_ref[...] = lax.dot_general(a, w_ref[...], dn,
                                 preferred_element_type=jnp.float32
                                 ) + b_ref[...]


def _wo(oa, ob, Wo, bo):
    gspec = pl.BlockSpec((HG, CH, 2 * HD), lambda c: (0, c, 0))
    return pl.pallas_call(
        _wo_body,
        grid=(NCH,),
        in_specs=[gspec, gspec,
                  pl.BlockSpec((DM, DM), lambda c: (0, 0)),
                  pl.BlockSpec((1, DM), lambda c: (0, 0))],
        out_specs=pl.BlockSpec((CH, DM), lambda c: (c, 0)),
        out_shape=jax.ShapeDtypeStruct((BS, DM), jnp.float32),
    )(oa, ob, Wo.astype(jnp.bfloat16), bo.reshape(1, DM))


def kernel(x, Wq, bq, Wk, bk, Wv, bv, Wo, bo, hash_proj):
    xf = x.reshape(BS, DM)
    qkvh, offs, counts = _proj_route(xf, Wq, bq, Wk, bk, Wv, bv, hash_proj)
    offs_flat = offs.reshape(NCHR * H * CHR)
    cnt = counts.reshape(H * NB)
    blk_a = _sc_scatter(qkvh, offs_flat, 0)
    blk_b = _sc_scatter(qkvh, offs_flat, HG)
    res_a = _attn(cnt[:HG * NB], blk_a)
    res_b = _attn(cnt[HG * NB:], blk_b)
    oa = _sc_gather(res_a, offs_flat, 0)
    ob = _sc_gather(res_b, offs_flat, HG)
    out = _wo(oa, ob, Wo, bo)
    return out.reshape(B, S, DM)


# speedup vs baseline: 1.1907x; 1.0253x over previous
import functools

import jax
import jax.numpy as jnp
from jax import lax
from jax.experimental import pallas as pl
from jax.experimental.pallas import tpu as pltpu
from jax.experimental.pallas import tpu_sc as plsc

B, S, DM = 2, 4096, 768
H, HD = 12, 64
NB, MAXB = 64, 256
BS = B * S
R = H * NB * MAXB
AB = 8
HG = H // 3
RG = HG * NB * MAXB
RGP = RG + AB * MAXB
SCALE = 1.0 / (HD ** 0.5)
CH = 1024
NCH = BS // CH
CHR = 512
NCHR = BS // CHR
PK = 2 * HD


def _bf16_hi_bits(x):
    rt = x.astype(jnp.bfloat16).astype(jnp.float32)
    return lax.bitcast_convert_type(rt, jnp.int32)


def _proj_route_body(x_ref, wq_ref, bq_ref, wk_ref, bk_ref, wv_ref, bv_ref,
                     hp_ref, qkv_ref, offs_ref, counts_ref, carry_ref):
    c = pl.program_id(0)
    dn2 = (((1,), (0,)), ((), ()))
    xb = x_ref[...]
    dn = (((1,), (1,)), ((), ()))
    yq = lax.dot_general(xb, wq_ref[...], dn) + bq_ref[...]
    xb16 = xb.astype(jnp.bfloat16)
    yk = lax.dot_general(xb16, wk_ref[...], dn,
                         preferred_element_type=jnp.float32) + bk_ref[...]
    yv = lax.dot_general(xb16, wv_ref[...], dn,
                         preferred_element_type=jnp.float32) + bv_ref[...]

    col = lax.broadcasted_iota(jnp.int32, (CHR, NB), 1).astype(jnp.float32)
    parts = []
    for h in range(H):
        hsb = jnp.dot(yq[:, h * HD:(h + 1) * HD], hp_ref[h])
        rowmax = jnp.max(hsb, axis=1, keepdims=True)
        am = jnp.min(jnp.where(hsb == rowmax, col, float(NB)),
                     axis=1, keepdims=True)
        parts.append((col == am).astype(jnp.float32))
    oneh = jnp.concatenate(parts, axis=1)

    ri = lax.broadcasted_iota(jnp.int32, (CHR, CHR), 0)
    ci = lax.broadcasted_iota(jnp.int32, (CHR, CHR), 1)
    ltri = (ri >= ci).astype(jnp.bfloat16)
    cum = lax.dot_general(ltri, oneh.astype(jnp.bfloat16), dn2,
                          preferred_element_type=jnp.float32)

    for h in range(H):
        oh = oneh[:, h * HD:(h + 1) * HD]
        cm = cum[:, h * HD:(h + 1) * HD]
        carry = jnp.where(c == 0, 0.0, carry_ref[h:h + 1, :])
        pos = jnp.sum((cm - 1.0 + carry) * oh, axis=1)
        b = jnp.sum(col * oh, axis=1)
        glob = (b + (h % HG) * NB) * MAXB + pos
        offs_ref[0, h, :] = jnp.where(pos < MAXB, glob,
                                      float(RG)).astype(jnp.int32)
        new_carry = carry + cm[CHR - 1:CHR, :]
        carry_ref[h:h + 1, :] = new_carry
        counts_ref[h:h + 1, :, :] = new_carry.reshape(1, 1, NB)

    for h in range(H):
        sl = slice(h * HD, (h + 1) * HD)
        hi = _bf16_hi_bits(jnp.concatenate([yq[:, sl], yv[:, sl]], axis=1))
        lo = _bf16_hi_bits(jnp.concatenate([yk[:, sl], yk[:, sl]], axis=1))
        qkv_ref[h] = hi | lax.shift_right_logical(lo, 16)


def _proj_route(xf, Wq, bq, Wk, bk, Wv, bv, hash_proj):
    wspec = pl.BlockSpec((DM, DM), lambda c: (0, 0))
    wspec16 = pl.BlockSpec((DM, DM), lambda c: (0, 0))
    bspec = pl.BlockSpec((1, DM), lambda c: (0, 0))
    rspec = pl.BlockSpec((CHR, DM), lambda c: (c, 0))
    return pl.pallas_call(
        _proj_route_body,
        grid=(NCHR,),
        in_specs=[rspec, wspec, bspec, wspec16, bspec, wspec16, bspec,
                  pl.BlockSpec((H, HD, NB), lambda c: (0, 0, 0))],
        out_specs=[
            pl.BlockSpec((H, CHR, PK), lambda c: (0, c, 0)),
            pl.BlockSpec((1, H, CHR), lambda c: (c, 0, 0)),
            pl.BlockSpec((H, 1, NB), lambda c: (0, 0, 0)),
        ],
        out_shape=[
            jax.ShapeDtypeStruct((H, BS, PK), jnp.int32),
            jax.ShapeDtypeStruct((NCHR, H, CHR), jnp.int32),
            jax.ShapeDtypeStruct((H, 1, NB), jnp.float32),
        ],
        scratch_shapes=[pltpu.VMEM((H, NB), jnp.float32)],
    )(xf, Wq, bq.reshape(1, DM), Wk.astype(jnp.bfloat16), bk.reshape(1, DM),
      Wv.astype(jnp.bfloat16), bv.reshape(1, DM), hash_proj)


def _sc_scatter(qkvh, offs_flat, h0):
    mesh = plsc.VectorSubcoreMesh(core_axis_name="c", subcore_axis_name="s")

    @functools.partial(
        pl.kernel,
        out_type=jax.ShapeDtypeStruct((RGP, PK), jnp.int32),
        mesh=mesh,
        scratch_types=[
            pltpu.VMEM((2, MAXB, PK), jnp.int32),
            pltpu.VMEM((2 * HG, 128), jnp.int32),
            pltpu.SemaphoreType.DMA,
            pltpu.SemaphoreType.DMA,
            pltpu.SemaphoreType.DMA,
            pltpu.SemaphoreType.DMA,
            pltpu.SemaphoreType.DMA,
        ],
    )
    def run(qkv_hbm, offs_hbm, blk_hbm, rows_v, idx_v, sem_i,
            sem_s0, sem_s1, sem_c0, sem_c1):
        wid = lax.axis_index("s") * 2 + lax.axis_index("c")
        t0 = wid * MAXB
        base0 = (t0 // CHR) * (H * CHR) + (t0 % CHR)
        sem_s = (sem_s0, sem_s1)
        sem_c = (sem_c0, sem_c1)

        iw = []
        for u, h in enumerate(range(h0, h0 + HG)):
            for j in range(2):
                iw.append(pltpu.async_copy(
                    offs_hbm.at[pl.ds(base0 + h * CHR + j * 128, 128)],
                    idx_v.at[2 * u + j], sem_i))
        for w in iw:
            w.wait()

        def stage(h, b):
            return pltpu.async_copy(qkv_hbm.at[h, pl.ds(t0, MAXB)],
                                    rows_v.at[b], sem_c[b])

        cw = stage(h0, 0)
        sw = []
        for u, h in enumerate(range(h0, h0 + HG)):
            b = u % 2
            cw.wait()
            if u >= 1:
                for w in sw:
                    w.wait()
            if u < HG - 1:
                cw = stage(h + 1, 1 - b)
            sw = [pltpu.async_copy(rows_v.at[b, pl.ds(j * 128, 128)],
                                   blk_hbm.at[idx_v.at[2 * u + j]],
                                   sem_s[b])
                  for j in range(2)]
        for w in sw:
            w.wait()

    return run(qkvh, offs_flat)


def _attn_body(cnt_ref, blk_ref, out_ref):
    g = pl.program_id(0)

    @pl.when(g == (HG * NB) // AB)
    def _zero():
        out_ref[...] = jnp.zeros_like(out_ref)

    @pl.when(g < (HG * NB) // AB)
    def _attend():
        rowi = lax.broadcasted_iota(jnp.int32, (MAXB, HD), 0).astype(
            jnp.float32)
        coli = lax.broadcasted_iota(jnp.int32, (MAXB, MAXB), 1).astype(
            jnp.float32)
        for i in range(AB):
            cnt = cnt_ref[g * AB + i]
            blk = blk_ref[i * MAXB:(i + 1) * MAXB, :]
            hi = lax.bitcast_convert_type(
                jnp.bitwise_and(blk, jnp.int32(-65536)), jnp.float32)
            lo = lax.bitcast_convert_type(
                lax.shift_left(blk, 16), jnp.float32)
            q = hi[:, 0:HD].astype(jnp.bfloat16)
            v = hi[:, HD:2 * HD]
            k = lo[:, 0:HD].astype(jnp.bfloat16)
            dots = lax.dot_general(q, k, (((1,), (1,)), ((), ())),
                                   preferred_element_type=jnp.float32)
            dots = dots * SCALE
            dots = jnp.where(coli < cnt, dots, -jnp.inf)
            e = jnp.exp(dots)
            s = jnp.sum(e, axis=1, keepdims=True)
            attnw = (e / s).astype(jnp.bfloat16)
            vb = jnp.where(rowi < cnt, v, 0.0).astype(jnp.bfloat16)
            res = lax.dot_general(attnw, vb, (((1,), (0,)), ((), ())),
                                  preferred_element_type=jnp.float32)
            out_ref[i * MAXB:(i + 1) * MAXB, :] = jnp.concatenate(
                [res, jnp.zeros((MAXB, HD), jnp.float32)], axis=1)


def _attn(cnt, qkv_blk):
    return pl.pallas_call(
        _attn_body,
        grid=((HG * NB) // AB + 1,),
        in_specs=[pl.BlockSpec(memory_space=pltpu.SMEM),
                  pl.BlockSpec((AB * MAXB, PK), lambda g: (g, 0))],
        out_specs=pl.BlockSpec((AB * MAXB, 2 * HD), lambda g: (g, 0)),
        out_shape=jax.ShapeDtypeStruct((RGP, 2 * HD), jnp.float32),
    )(cnt, qkv_blk)


def _sc_gather(res, offs_flat, h0):
    mesh = plsc.VectorSubcoreMesh(core_axis_name="c", subcore_axis_name="s")

    @functools.partial(
        pl.kernel,
        out_type=jax.ShapeDtypeStruct((HG, BS, 2 * HD), jnp.float32),
        mesh=mesh,
        scratch_types=[
            pltpu.VMEM((2, 128, 2 * HD), jnp.float32),
            pltpu.VMEM((2 * HG, 128), jnp.int32),
            pltpu.SemaphoreType.DMA,
            pltpu.SemaphoreType.DMA,
            pltpu.SemaphoreType.DMA,
            pltpu.SemaphoreType.DMA,
            pltpu.SemaphoreType.DMA,
        ],
    )
    def run(res_hbm, offs_hbm, out_hbm, gbuf, idxg, sem_i,
            sem_g0, sem_g1, sem_w0, sem_w1):
        wid = lax.axis_index("s") * 2 + lax.axis_index("c")
        sem_g = (sem_g0, sem_g1)
        sem_w = (sem_w0, sem_w1)
        units = [((wid * 2 + half) * 128, h)
                 for half in range(2) for h in range(h0, h0 + HG)]

        iw = []
        for u, (t0, h) in enumerate(units):
            base = (t0 // CHR) * (H * CHR) + (t0 % CHR) + h * CHR
            iw.append(pltpu.async_copy(offs_hbm.at[pl.ds(base, 128)],
                                       idxg.at[u], sem_i))
        for w in iw:
            w.wait()

        def fire(u, b):
            return pltpu.async_copy(res_hbm.at[idxg.at[u]], gbuf.at[b],
                                    sem_g[b])

        gw = fire(0, 0)
        ww = None
        for u, (t0, h) in enumerate(units):
            b = u % 2
            gw.wait()
            if ww is not None:
                ww.wait()
            if u < len(units) - 1:
                gw = fire(u + 1, 1 - b)
            ww = pltpu.async_copy(gbuf.at[b],
                                  out_hbm.at[h - h0, pl.ds(t0, 128)],
                                  sem_w[b])
        ww.wait()

    return run(res, offs_flat)


def _wo_body(oa_ref, ob_ref, oc_ref, w_ref, b_ref, o_ref):
    a = jnp.concatenate([g[h][:, :HD]
                         for g in (oa_ref, ob_ref, oc_ref)
                         for h in range(HG)],
                        axis=1).astype(jnp.bfloat16)
    dn = (((1,), (1,)), ((), ()))
    o_ref[...] = lax.dot_general(a, w_ref[...], dn,
                                 preferred_element_type=jnp.float32
                                 ) + b_ref[...]


def _wo(oa, ob, oc, Wo, bo):
    gspec = pl.BlockSpec((HG, CH, 2 * HD), lambda c: (0, c, 0))
    return pl.pallas_call(
        _wo_body,
        grid=(NCH,),
        in_specs=[gspec, gspec, gspec,
                  pl.BlockSpec((DM, DM), lambda c: (0, 0)),
                  pl.BlockSpec((1, DM), lambda c: (0, 0))],
        out_specs=pl.BlockSpec((CH, DM), lambda c: (c, 0)),
        out_shape=jax.ShapeDtypeStruct((BS, DM), jnp.float32),
    )(oa, ob, oc, Wo.astype(jnp.bfloat16), bo.reshape(1, DM))


def kernel(x, Wq, bq, Wk, bk, Wv, bv, Wo, bo, hash_proj):
    xf = x.reshape(BS, DM)
    qkvh, offs, counts = _proj_route(xf, Wq, bq, Wk, bk, Wv, bv, hash_proj)
    offs_flat = offs.reshape(NCHR * H * CHR)
    cnt = counts.reshape(H * NB)
    blk_a = _sc_scatter(qkvh, offs_flat, 0)
    blk_b = _sc_scatter(qkvh, offs_flat, HG)
    blk_c = _sc_scatter(qkvh, offs_flat, 2 * HG)
    res_a = _attn(cnt[:HG * NB], blk_a)
    res_b = _attn(cnt[HG * NB:2 * HG * NB], blk_b)
    res_c = _attn(cnt[2 * HG * NB:], blk_c)
    oa = _sc_gather(res_a, offs_flat, 0)
    ob = _sc_gather(res_b, offs_flat, HG)
    oc = _sc_gather(res_c, offs_flat, 2 * HG)
    out = _wo(oa, ob, oc, Wo, bo)
    return out.reshape(B, S, DM)


# speedup vs baseline: 1.2305x; 1.0334x over previous
import functools

import jax
import jax.numpy as jnp
from jax import lax
from jax.experimental import pallas as pl
from jax.experimental.pallas import tpu as pltpu
from jax.experimental.pallas import tpu_sc as plsc

B, S, DM = 2, 4096, 768
H, HD = 12, 64
NB, MAXB = 64, 256
BS = B * S
R = H * NB * MAXB
AB = 8
HG = H // 3
RG = HG * NB * MAXB
RGP = RG + AB * MAXB
SCALE = 1.0 / (HD ** 0.5)
CH = 1024
NCH = BS // CH
CHR = 512
NCHR = BS // CHR
PK = 2 * HD


def _bf16_hi_bits(x):
    rt = x.astype(jnp.bfloat16).astype(jnp.float32)
    return lax.bitcast_convert_type(rt, jnp.int32)


def _proj_route_body(x_ref, wq_ref, bq_ref, wk_ref, bk_ref, wv_ref, bv_ref,
                     hp_ref, qkv_ref, offs_ref, counts_ref, carry_ref):
    c = pl.program_id(0)
    dn2 = (((1,), (0,)), ((), ()))
    xb = x_ref[...]
    dn = (((1,), (1,)), ((), ()))
    yq = lax.dot_general(xb, wq_ref[...], dn) + bq_ref[...]
    xb16 = xb.astype(jnp.bfloat16)
    yk = lax.dot_general(xb16, wk_ref[...], dn,
                         preferred_element_type=jnp.float32) + bk_ref[...]
    yv = lax.dot_general(xb16, wv_ref[...], dn,
                         preferred_element_type=jnp.float32) + bv_ref[...]

    col = lax.broadcasted_iota(jnp.int32, (CHR, NB), 1).astype(jnp.float32)
    parts = []
    for h in range(H):
        hsb = jnp.dot(yq[:, h * HD:(h + 1) * HD], hp_ref[h])
        rowmax = jnp.max(hsb, axis=1, keepdims=True)
        am = jnp.min(jnp.where(hsb == rowmax, col, float(NB)),
                     axis=1, keepdims=True)
        parts.append((col == am).astype(jnp.float32))
    oneh = jnp.concatenate(parts, axis=1)

    ri = lax.broadcasted_iota(jnp.int32, (CHR, CHR), 0)
    ci = lax.broadcasted_iota(jnp.int32, (CHR, CHR), 1)
    ltri = (ri >= ci).astype(jnp.bfloat16)
    cum = lax.dot_general(ltri, oneh.astype(jnp.bfloat16), dn2,
                          preferred_element_type=jnp.float32)

    for h in range(H):
        oh = oneh[:, h * HD:(h + 1) * HD]
        cm = cum[:, h * HD:(h + 1) * HD]
        carry = jnp.where(c == 0, 0.0, carry_ref[h:h + 1, :])
        pos = jnp.sum((cm - 1.0 + carry) * oh, axis=1,
                      keepdims=True)
        b = jnp.sum(col * oh, axis=1, keepdims=True)
        glob = (b + (h % HG) * NB) * MAXB + pos
        offs_ref[:, h:h + 1] = jnp.where(pos < MAXB, glob,
                                         float(RG)).astype(jnp.int32)
        new_carry = carry + cm[CHR - 1:CHR, :]
        carry_ref[h:h + 1, :] = new_carry
        counts_ref[h:h + 1, :, :] = new_carry.reshape(1, 1, NB)

    for h in range(H):
        sl = slice(h * HD, (h + 1) * HD)
        hi = _bf16_hi_bits(jnp.concatenate([yq[:, sl], yv[:, sl]], axis=1))
        lo = _bf16_hi_bits(jnp.concatenate([yk[:, sl], yk[:, sl]], axis=1))
        qkv_ref[h] = hi | lax.shift_right_logical(lo, 16)


def _proj_route(xf, Wq, bq, Wk, bk, Wv, bv, hash_proj):
    wspec = pl.BlockSpec((DM, DM), lambda c: (0, 0))
    wspec16 = pl.BlockSpec((DM, DM), lambda c: (0, 0))
    bspec = pl.BlockSpec((1, DM), lambda c: (0, 0))
    rspec = pl.BlockSpec((CHR, DM), lambda c: (c, 0))
    return pl.pallas_call(
        _proj_route_body,
        grid=(NCHR,),
        in_specs=[rspec, wspec, bspec, wspec16, bspec, wspec16, bspec,
                  pl.BlockSpec((H, HD, NB), lambda c: (0, 0, 0))],
        out_specs=[
            pl.BlockSpec((H, CHR, PK), lambda c: (0, c, 0)),
            pl.BlockSpec((CHR, H), lambda c: (c, 0)),
            pl.BlockSpec((H, 1, NB), lambda c: (0, 0, 0)),
        ],
        out_shape=[
            jax.ShapeDtypeStruct((H, BS, PK), jnp.int32),
            jax.ShapeDtypeStruct((BS, H), jnp.int32),
            jax.ShapeDtypeStruct((H, 1, NB), jnp.float32),
        ],
        scratch_shapes=[pltpu.VMEM((H, NB), jnp.float32)],
    )(xf, Wq, bq.reshape(1, DM), Wk.astype(jnp.bfloat16), bk.reshape(1, DM),
      Wv.astype(jnp.bfloat16), bv.reshape(1, DM), hash_proj)


def _sc_scatter(qkvh, offs_flat, h0):
    mesh = plsc.VectorSubcoreMesh(core_axis_name="c", subcore_axis_name="s")

    @functools.partial(
        pl.kernel,
        out_type=jax.ShapeDtypeStruct((RGP, PK), jnp.int32),
        mesh=mesh,
        scratch_types=[
            pltpu.VMEM((2, MAXB, PK), jnp.int32),
            pltpu.VMEM((2 * HG, 128), jnp.int32),
            pltpu.SemaphoreType.DMA,
            pltpu.SemaphoreType.DMA,
            pltpu.SemaphoreType.DMA,
            pltpu.SemaphoreType.DMA,
            pltpu.SemaphoreType.DMA,
        ],
    )
    def run(qkv_hbm, offs_hbm, blk_hbm, rows_v, idx_v, sem_i,
            sem_s0, sem_s1, sem_c0, sem_c1):
        wid = lax.axis_index("s") * 2 + lax.axis_index("c")
        t0 = wid * MAXB
        sem_s = (sem_s0, sem_s1)
        sem_c = (sem_c0, sem_c1)

        iw = []
        for u, h in enumerate(range(h0, h0 + HG)):
            for j in range(2):
                iw.append(pltpu.async_copy(
                    offs_hbm.at[pl.ds(h * BS + t0 + j * 128, 128)],
                    idx_v.at[2 * u + j], sem_i))
        for w in iw:
            w.wait()

        def stage(h, b):
            return pltpu.async_copy(qkv_hbm.at[h, pl.ds(t0, MAXB)],
                                    rows_v.at[b], sem_c[b])

        cw = stage(h0, 0)
        sw = []
        for u, h in enumerate(range(h0, h0 + HG)):
            b = u % 2
            cw.wait()
            if u >= 1:
                for w in sw:
                    w.wait()
            if u < HG - 1:
                cw = stage(h + 1, 1 - b)
            sw = [pltpu.async_copy(rows_v.at[b, pl.ds(j * 128, 128)],
                                   blk_hbm.at[idx_v.at[2 * u + j]],
                                   sem_s[b])
                  for j in range(2)]
        for w in sw:
            w.wait()

    return run(qkvh, offs_flat)


def _attn_body(cnt_ref, blk_ref, out_ref):
    g = pl.program_id(0)

    @pl.when(g == (HG * NB) // AB)
    def _zero():
        out_ref[...] = jnp.zeros_like(out_ref)

    @pl.when(g < (HG * NB) // AB)
    def _attend():
        rowi = lax.broadcasted_iota(jnp.int32, (MAXB, HD), 0).astype(
            jnp.float32)
        coli = lax.broadcasted_iota(jnp.int32, (MAXB, MAXB), 1).astype(
            jnp.float32)
        for i in range(AB):
            cnt = cnt_ref[g * AB + i]
            blk = blk_ref[i * MAXB:(i + 1) * MAXB, :]
            hi = lax.bitcast_convert_type(
                jnp.bitwise_and(blk, jnp.int32(-65536)), jnp.float32)
            lo = lax.bitcast_convert_type(
                lax.shift_left(blk, 16), jnp.float32)
            q = hi[:, 0:HD].astype(jnp.bfloat16)
            v = hi[:, HD:2 * HD]
            k = lo[:, 0:HD].astype(jnp.bfloat16)
            dots = lax.dot_general(q, k, (((1,), (1,)), ((), ())),
                                   preferred_element_type=jnp.float32)
            dots = dots * SCALE
            dots = jnp.where(coli < cnt, dots, -jnp.inf)
            e = jnp.exp(dots)
            s = jnp.sum(e, axis=1, keepdims=True)
            attnw = (e / s).astype(jnp.bfloat16)
            vb = jnp.where(rowi < cnt, v, 0.0).astype(jnp.bfloat16)
            res = lax.dot_general(attnw, vb, (((1,), (0,)), ((), ())),
                                  preferred_element_type=jnp.float32)
            out_ref[i * MAXB:(i + 1) * MAXB, :] = jnp.concatenate(
                [res, jnp.zeros((MAXB, HD), jnp.float32)], axis=1)


def _attn(cnt, qkv_blk):
    return pl.pallas_call(
        _attn_body,
        grid=((HG * NB) // AB + 1,),
        in_specs=[pl.BlockSpec(memory_space=pltpu.SMEM),
                  pl.BlockSpec((AB * MAXB, PK), lambda g: (g, 0))],
        out_specs=pl.BlockSpec((AB * MAXB, 2 * HD), lambda g: (g, 0)),
        out_shape=jax.ShapeDtypeStruct((RGP, 2 * HD), jnp.float32),
    )(cnt, qkv_blk)


def _sc_gather(res, offs_flat, h0):
    mesh = plsc.VectorSubcoreMesh(core_axis_name="c", subcore_axis_name="s")

    @functools.partial(
        pl.kernel,
        out_type=jax.ShapeDtypeStruct((HG, BS, 2 * HD), jnp.float32),
        mesh=mesh,
        scratch_types=[
            pltpu.VMEM((2, 128, 2 * HD), jnp.float32),
            pltpu.VMEM((2 * HG, 128), jnp.int32),
            pltpu.SemaphoreType.DMA,
            pltpu.SemaphoreType.DMA,
            pltpu.SemaphoreType.DMA,
            pltpu.SemaphoreType.DMA,
            pltpu.SemaphoreType.DMA,
        ],
    )
    def run(res_hbm, offs_hbm, out_hbm, gbuf, idxg, sem_i,
            sem_g0, sem_g1, sem_w0, sem_w1):
        wid = lax.axis_index("s") * 2 + lax.axis_index("c")
        sem_g = (sem_g0, sem_g1)
        sem_w = (sem_w0, sem_w1)
        units = [((wid * 2 + half) * 128, h)
                 for half in range(2) for h in range(h0, h0 + HG)]

        iw = []
        for u, (t0, h) in enumerate(units):
            iw.append(pltpu.async_copy(offs_hbm.at[pl.ds(h * BS + t0, 128)],
                                       idxg.at[u], sem_i))
        for w in iw:
            w.wait()

        def fire(u, b):
            return pltpu.async_copy(res_hbm.at[idxg.at[u]], gbuf.at[b],
                                    sem_g[b])

        gw = fire(0, 0)
        ww = None
        for u, (t0, h) in enumerate(units):
            b = u % 2
            gw.wait()
            if ww is not None:
                ww.wait()
            if u < len(units) - 1:
                gw = fire(u + 1, 1 - b)
            ww = pltpu.async_copy(gbuf.at[b],
                                  out_hbm.at[h - h0, pl.ds(t0, 128)],
                                  sem_w[b])
        ww.wait()

    return run(res, offs_flat)


def _wo_body(oa_ref, ob_ref, oc_ref, w_ref, b_ref, o_ref):
    a = jnp.concatenate([g[h][:, :HD]
                         for g in (oa_ref, ob_ref, oc_ref)
                         for h in range(HG)],
                        axis=1).astype(jnp.bfloat16)
    dn = (((1,), (1,)), ((), ()))
    o_ref[...] = lax.dot_general(a, w_ref[...], dn,
                                 preferred_element_type=jnp.float32
                                 ) + b_ref[...]


def _wo(oa, ob, oc, Wo, bo):
    gspec = pl.BlockSpec((HG, CH, 2 * HD), lambda c: (0, c, 0))
    return pl.pallas_call(
        _wo_body,
        grid=(NCH,),
        in_specs=[gspec, gspec, gspec,
                  pl.BlockSpec((DM, DM), lambda c: (0, 0)),
                  pl.BlockSpec((1, DM), lambda c: (0, 0))],
        out_specs=pl.BlockSpec((CH, DM), lambda c: (c, 0)),
        out_shape=jax.ShapeDtypeStruct((BS, DM), jnp.float32),
    )(oa, ob, oc, Wo.astype(jnp.bfloat16), bo.reshape(1, DM))


def kernel(x, Wq, bq, Wk, bk, Wv, bv, Wo, bo, hash_proj):
    xf = x.reshape(BS, DM)
    qkvh, offs, counts = _proj_route(xf, Wq, bq, Wk, bk, Wv, bv, hash_proj)
    offs_flat = offs.T.reshape(H * BS)
    cnt = counts.reshape(H * NB)
    blk_a = _sc_scatter(qkvh, offs_flat, 0)
    blk_b = _sc_scatter(qkvh, offs_flat, HG)
    blk_c = _sc_scatter(qkvh, offs_flat, 2 * HG)
    res_a = _attn(cnt[:HG * NB], blk_a)
    res_b = _attn(cnt[HG * NB:2 * HG * NB], blk_b)
    res_c = _attn(cnt[2 * HG * NB:], blk_c)
    oa = _sc_gather(res_a, offs_flat, 0)
    ob = _sc_gather(res_b, offs_flat, HG)
    oc = _sc_gather(res_c, offs_flat, 2 * HG)
    out = _wo(oa, ob, oc, Wo, bo)
    return out.reshape(B, S, DM)
